# baseline (speedup 1.0000x reference)
import functools

import jax
import jax.numpy as jnp
from jax import lax
from jax.experimental import pallas as pl
from jax.experimental.pallas import tpu as pltpu
from jax.experimental.pallas import tpu_sc as plsc

_B, _Y, _Z = 128, 2048, 128


def _make_gather():
    info = plsc.get_sparse_core_info()
    nc = info.num_cores
    n_active = 16
    rows_per_w = _B // n_active
    mesh = plsc.VectorSubcoreMesh(core_axis_name="c", subcore_axis_name="s")

    @functools.partial(
        pl.kernel,
        mesh=mesh,
        out_type=jax.ShapeDtypeStruct((_B, _Z), jnp.float32),
        scratch_types=[
            pltpu.VMEM((rows_per_w,), jnp.int32),
            pltpu.VMEM((rows_per_w, _Z), jnp.float32),
            pltpu.SemaphoreType.DMA,
        ],
    )
    def gather_kernel(table_hbm, idx_hbm, out_hbm, idx_v, rows_v, sem):
        wid = lax.axis_index("s") * nc + lax.axis_index("c")

        @pl.when(wid < n_active)
        def _():
            base = wid * rows_per_w
            pltpu.sync_copy(idx_hbm.at[pl.ds(base, rows_per_w)], idx_v)
            pltpu.async_copy(table_hbm.at[idx_v], rows_v, sem).wait()
            pltpu.sync_copy(rows_v, out_hbm.at[pl.ds(base, rows_per_w)])

    return gather_kernel


_gather = _make_gather()


@jax.jit
def kernel(x):
    B, Y, Z = x.shape
    idx_key = jax.random.fold_in(jax.random.key(0), 1)
    randomind = jax.random.randint(idx_key, (B,), 0, Y, dtype=jnp.int32)
    flat_idx = jnp.arange(B, dtype=jnp.int32) * Y + randomind
    table = x.reshape(B * Y, Z)
    return _gather(table, flat_idx)


# speedup vs baseline: 1.0546x; 1.0546x over previous
import functools

import jax
import jax.numpy as jnp
from jax import lax
from jax.experimental import pallas as pl
from jax.experimental.pallas import tpu as pltpu
from jax.experimental.pallas import tpu_sc as plsc

_B, _Y, _Z = 128, 2048, 128


def _make_gather():
    n_workers = 16
    rows_per_w = _B // n_workers
    mesh = plsc.VectorSubcoreMesh(
        core_axis_name="c", subcore_axis_name="s", num_cores=1
    )

    @functools.partial(
        pl.kernel,
        mesh=mesh,
        out_type=jax.ShapeDtypeStruct((_B, _Z), jnp.float32),
        scratch_types=[
            pltpu.VMEM((rows_per_w,), jnp.int32),
            pltpu.VMEM((rows_per_w, _Z), jnp.float32),
            pltpu.SemaphoreType.DMA,
        ],
    )
    def gather_kernel(table_hbm, idx_hbm, out_hbm, idx_v, rows_v, sem):
        wid = lax.axis_index("s")
        base = wid * rows_per_w
        pltpu.sync_copy(idx_hbm.at[pl.ds(base, rows_per_w)], idx_v)
        pltpu.async_copy(table_hbm.at[idx_v], rows_v, sem).wait()
        pltpu.sync_copy(rows_v, out_hbm.at[pl.ds(base, rows_per_w)])

    return gather_kernel


_gather = _make_gather()


@jax.jit
def kernel(x):
    B, Y, Z = x.shape
    idx_key = jax.random.fold_in(jax.random.key(0), 1)
    randomind = jax.random.randint(idx_key, (B,), 0, Y, dtype=jnp.int32)
    flat_idx = jnp.arange(B, dtype=jnp.int32) * Y + randomind
    table = x.reshape(B * Y, Z)
    return _gather(table, flat_idx)


# speedup vs baseline: 1.1827x; 1.1215x over previous
import functools

import jax
import jax.numpy as jnp
import numpy as np
from jax import lax
from jax.experimental import pallas as pl
from jax.experimental.pallas import tpu as pltpu
from jax.experimental.pallas import tpu_sc as plsc

_B, _Y, _Z = 128, 2048, 128


def _flat_indices() -> np.ndarray:
    idx_key = jax.random.fold_in(jax.random.key(0), 1)
    t = jax.random.randint(idx_key, (_B,), 0, _Y, dtype=jnp.int32)
    return np.arange(_B, dtype=np.int32) * _Y + np.asarray(t)


_FLAT_IDX = _flat_indices()


def _make_gather():
    n_workers = 16
    rows_per_w = _B // n_workers
    mesh = plsc.VectorSubcoreMesh(
        core_axis_name="c", subcore_axis_name="s", num_cores=1
    )

    @functools.partial(
        pl.kernel,
        mesh=mesh,
        out_type=jax.ShapeDtypeStruct((_B, _Z), jnp.float32),
        scratch_types=[
            pltpu.VMEM((rows_per_w,), jnp.int32),
            pltpu.VMEM((rows_per_w, _Z), jnp.float32),
            pltpu.SemaphoreType.DMA,
        ],
    )
    def gather_kernel(table_hbm, idx_hbm, out_hbm, idx_v, rows_v, sem):
        wid = lax.axis_index("s")
        base = wid * rows_per_w
        pltpu.sync_copy(idx_hbm.at[pl.ds(base, rows_per_w)], idx_v)
        pltpu.async_copy(table_hbm.at[idx_v], rows_v, sem).wait()
        pltpu.sync_copy(rows_v, out_hbm.at[pl.ds(base, rows_per_w)])

    return gather_kernel


_gather = _make_gather()


@jax.jit
def kernel(x):
    B, Y, Z = x.shape
    flat_idx = jnp.asarray(_FLAT_IDX)
    table = x.reshape(B * Y, Z)
    return _gather(table, flat_idx)
